# baseline (speedup 1.0000x reference)
import functools

import jax
import jax.numpy as jnp
from jax import lax
from jax.experimental import pallas as pl
from jax.experimental.pallas import tpu as pltpu
from jax.experimental.pallas import tpu_sc as plsc

_NC = 2
_NS = 16
_NW = _NC * _NS
_C = 1600


def _rsqrt16(d):
    i = lax.bitcast_convert_type(d, jnp.int32)
    i = jnp.int32(0x5F3759DF) - lax.shift_right_logical(i, 1)
    y = lax.bitcast_convert_type(i, jnp.float32)
    for _ in range(3):
        y = y * (1.5 - 0.5 * d * y * y)
    return y


def _make_pass(n_pad, e_pad, mode):
    k_total = e_pad // (_NW * _C)
    assert k_total % 2 == 0
    k2 = k_total // 2
    ew = k_total * _C
    slc = n_pad // _NS
    has_g = mode != "deg"
    nsc = 2 if mode == "pm" else 1
    n_tab_in = {"deg": 0, "sum": 2, "pm": 3}[mode]

    def buf_set():
        t = []
        if has_g:
            t.append(pltpu.VMEM((_C,), jnp.int32))
        t.append(pltpu.VMEM((_C,), jnp.int32))
        t.append(pltpu.VMEM((_C,), jnp.float32))
        if has_g:
            t.append(pltpu.VMEM((_C,), jnp.float32))
            for _ in range(nsc):
                t.append(pltpu.VMEM((_C,), jnp.float32))
        return t

    nbuf = len(buf_set())
    naux = {"deg": 0, "sum": 3, "pm": 5}[mode]
    scratch_types = buf_set() + buf_set()
    scratch_types.append(pltpu.VMEM((slc,), jnp.float32))
    for _ in range(naux):
        scratch_types.append(pltpu.VMEM((slc,), jnp.float32))
    if has_g:
        scratch_types.append(pltpu.VMEM_SHARED((n_pad,), jnp.float32))
    for _ in range(nsc):
        scratch_types.append(pltpu.VMEM_SHARED((n_pad,), jnp.float32))
    scratch_types += [pltpu.SemaphoreType.DMA] * 4

    out_sds = jax.ShapeDtypeStruct((_NC * n_pad,), jnp.float32)
    out_type = tuple(out_sds for _ in range(nsc)) if nsc > 1 else out_sds

    @functools.partial(
        pl.kernel,
        mesh=plsc.VectorSubcoreMesh(core_axis_name="c", subcore_axis_name="s"),
        out_type=out_type,
        scratch_types=scratch_types,
    )
    def edge_pass(*refs):
        n_in = 3 + n_tab_in if has_g else 2
        ins = refs[:n_in]
        edge_ins = ins[n_tab_in:]
        if has_g:
            src_hbm, dst_hbm, w_hbm = edge_ins
        else:
            dst_hbm, w_hbm = edge_ins
        outs = refs[n_in:n_in + nsc]
        sc = refs[n_in + nsc:]
        bufa = sc[:nbuf]
        bufb = sc[nbuf:2 * nbuf]
        tmp_v = sc[2 * nbuf]
        aux = sc[2 * nbuf + 1:2 * nbuf + 1 + naux]
        pos = 2 * nbuf + 1 + naux
        if has_g:
            tab_sh = sc[pos]
            pos += 1
        accs = sc[pos:pos + nsc]
        in_sema, in_semb, sc_sema, sc_semb = sc[pos + nsc:pos + nsc + 4]

        def parts(buf):
            if has_g:
                return {"src": buf[0], "dst": buf[1], "w": buf[2],
                        "g": buf[3], "v": buf[4:4 + nsc]}
            return {"dst": buf[0], "w": buf[1], "v": [buf[1]]}

        ba, bb = parts(bufa), parts(bufb)

        c = lax.axis_index("c")
        s = lax.axis_index("s")
        wid = c * _NS + s
        base_w = wid * ew

        def zero_body(j, carry):
            tmp_v[pl.ds(j * 16, 16)] = jnp.zeros((16,), jnp.float32)
            return carry

        lax.fori_loop(0, slc // 16, zero_body, 0)
        for acc in accs:
            pltpu.sync_copy(tmp_v, acc.at[pl.ds(s * slc, slc)])

        if mode == "sum":
            deg_hbm, x_hbm = ins[0], ins[1]
            qa, qb, qc = aux
            pltpu.sync_copy(deg_hbm.at[pl.ds(s * slc, slc)], qa)
            pltpu.sync_copy(deg_hbm.at[pl.ds(n_pad + s * slc, slc)], qb)
            pltpu.sync_copy(x_hbm.at[pl.ds(s * slc, slc)], qc)

            def tab_body(j, carry):
                o = j * 16
                deg = qa[pl.ds(o, 16)] + qb[pl.ds(o, 16)] + 1.0
                tmp_v[pl.ds(o, 16)] = qc[pl.ds(o, 16)] * _rsqrt16(deg)
                return carry

            lax.fori_loop(0, slc // 16, tab_body, 0)
            pltpu.sync_copy(tmp_v, tab_sh.at[pl.ds(s * slc, slc)])
        elif mode == "pm":
            deg_hbm, sacc_hbm, x_hbm = ins[0], ins[1], ins[2]
            qa, qb, qc, qd, qe = aux
            pltpu.sync_copy(deg_hbm.at[pl.ds(s * slc, slc)], qa)
            pltpu.sync_copy(deg_hbm.at[pl.ds(n_pad + s * slc, slc)], qb)
            pltpu.sync_copy(x_hbm.at[pl.ds(s * slc, slc)], qc)
            pltpu.sync_copy(sacc_hbm.at[pl.ds(s * slc, slc)], qd)
            pltpu.sync_copy(sacc_hbm.at[pl.ds(n_pad + s * slc, slc)], qe)

            def tab_body(j, carry):
                o = j * 16
                deg = qa[pl.ds(o, 16)] + qb[pl.ds(o, 16)] + 1.0
                y = _rsqrt16(deg)
                acc16 = qd[pl.ds(o, 16)] + qe[pl.ds(o, 16)]
                tmp_v[pl.ds(o, 16)] = (
                    y * y * (acc16 + qc[pl.ds(o, 16)] * y))
                return carry

            lax.fori_loop(0, slc // 16, tab_body, 0)
            pltpu.sync_copy(tmp_v, tab_sh.at[pl.ds(s * slc, slc)])
        plsc.subcore_barrier()

        def start_in(k, b, sem):
            base = base_w + k * _C
            if has_g:
                pltpu.async_copy(src_hbm.at[pl.ds(base, _C)], b["src"], sem)
            pltpu.async_copy(dst_hbm.at[pl.ds(base, _C)], b["dst"], sem)
            pltpu.async_copy(w_hbm.at[pl.ds(base, _C)], b["w"], sem)

        def wait_in(b, sem):
            if has_g:
                pltpu.make_async_copy(
                    src_hbm.at[pl.ds(0, _C)], b["src"], sem).wait()
            pltpu.make_async_copy(dst_hbm.at[pl.ds(0, _C)], b["dst"], sem).wait()
            pltpu.make_async_copy(w_hbm.at[pl.ds(0, _C)], b["w"], sem).wait()

        def compute(b):
            if mode == "sum":
                def body(j, carry):
                    o = j * 16
                    b["v"][0][pl.ds(o, 16)] = (
                        b["w"][pl.ds(o, 16)] * b["g"][pl.ds(o, 16)])
                    return carry
                lax.fori_loop(0, _C // 16, body, 0)
            elif mode == "pm":
                def body(j, carry):
                    o = j * 16
                    g = b["g"][pl.ds(o, 16)]
                    w16 = b["w"][pl.ds(o, 16)]
                    b["v"][0][pl.ds(o, 16)] = w16 * jnp.maximum(g, 0.0)
                    b["v"][1][pl.ds(o, 16)] = w16 * jnp.minimum(g, 0.0)
                    return carry
                lax.fori_loop(0, _C // 16, body, 0)

        def start_sc(b, sem):
            for v, acc in zip(b["v"], accs):
                pltpu.async_copy(v, acc.at[b["dst"]], sem, add=True)

        def wait_sc(b, sem):
            for v, acc in zip(b["v"], accs):
                pltpu.make_async_copy(v, acc.at[b["dst"]], sem).wait()

        def process(b, in_sem, sc_sem):
            wait_in(b, in_sem)
            if has_g:
                pltpu.sync_copy(tab_sh.at[b["src"]], b["g"])
            compute(b)
            start_sc(b, sc_sem)

        start_in(0, ba, in_sema)
        start_in(1, bb, in_semb)

        def pair_body(kk, carry):
            process(ba, in_sema, sc_sema)
            process(bb, in_semb, sc_semb)

            @pl.when(kk + 1 < k2)
            def _():
                wait_sc(ba, sc_sema)
                start_in(2 * kk + 2, ba, in_sema)
                wait_sc(bb, sc_semb)
                start_in(2 * kk + 3, bb, in_semb)

            return carry

        lax.fori_loop(0, k2, pair_body, 0)
        wait_sc(ba, sc_sema)
        wait_sc(bb, sc_semb)
        plsc.subcore_barrier()
        for acc, out in zip(accs, outs):
            pltpu.sync_copy(acc.at[pl.ds(s * slc, slc)], tmp_v)
            pltpu.sync_copy(tmp_v, out.at[pl.ds(c * n_pad + s * slc, slc)])

    return edge_pass


def _final_kernel(dp_ref, sacc_ref, cp_ref, cm_ref, x_ref,
                  w1_ref, w2_ref, b2_ref, out_ref):
    hdim = w2_ref.shape[1]
    deg = dp_ref[0] + dp_ref[1] + 1.0
    d = lax.rsqrt(deg)
    xd = x_ref[...] * d
    gs = d * d * (sacc_ref[0] + sacc_ref[1] + xd)
    alpha = d * (cp_ref[0] + cp_ref[1] + jnp.maximum(gs, 0.0))
    beta = d * (cm_ref[0] + cm_ref[1] + jnp.minimum(gs, 0.0))
    for h in range(hdim):
        up_h = 0.0
        um_h = 0.0
        for k in range(w2_ref.shape[0]):
            w1k = w1_ref[0, k]
            w2kh = w2_ref[k, h]
            up_h = up_h + jnp.maximum(w1k, 0.0) * w2kh
            um_h = um_h + jnp.minimum(w1k, 0.0) * w2kh
        out_ref[h] = alpha * up_h + beta * um_h + b2_ref[0, h]


def kernel(x, edge_index, edge_weight, W1, b1, W2, b2):
    n = x.shape[0]
    e = edge_weight.shape[0]
    hdim = W2.shape[1]

    n_pad = -(-n // 128) * 128
    group = _NW * _C * 2
    e_pad = -(-e // group) * group
    rows = n_pad // 128

    src = edge_index[0].astype(jnp.int32)
    dst = edge_index[1].astype(jnp.int32)
    w = edge_weight.astype(jnp.float32)
    npad_e = e_pad - e
    if npad_e:
        pad_idx = jnp.arange(npad_e, dtype=jnp.int32) % jnp.int32(n)
        src = jnp.concatenate([src, pad_idx])
        dst = jnp.concatenate([dst, pad_idx])
        w = jnp.concatenate([w, jnp.zeros((npad_e,), jnp.float32)])

    x1 = jnp.pad(x[:, 0].astype(jnp.float32), (0, n_pad - n))

    deg_p = _make_pass(n_pad, e_pad, "deg")(dst, w)

    sacc = _make_pass(n_pad, e_pad, "sum")(deg_p, x1, src, dst, w)

    cp, cm = _make_pass(n_pad, e_pad, "pm")(deg_p, sacc, x1, src, dst, w)

    out3 = pl.pallas_call(
        _final_kernel,
        in_specs=[
            pl.BlockSpec(memory_space=pltpu.VMEM),
            pl.BlockSpec(memory_space=pltpu.VMEM),
            pl.BlockSpec(memory_space=pltpu.VMEM),
            pl.BlockSpec(memory_space=pltpu.VMEM),
            pl.BlockSpec(memory_space=pltpu.VMEM),
            pl.BlockSpec(memory_space=pltpu.SMEM),
            pl.BlockSpec(memory_space=pltpu.SMEM),
            pl.BlockSpec(memory_space=pltpu.SMEM),
        ],
        out_shape=jax.ShapeDtypeStruct((hdim, rows, 128), jnp.float32),
    )(deg_p.reshape(_NC, rows, 128), sacc.reshape(_NC, rows, 128),
      cp.reshape(_NC, rows, 128), cm.reshape(_NC, rows, 128),
      x1.reshape(rows, 128),
      W1.astype(jnp.float32), W2.astype(jnp.float32),
      b2.astype(jnp.float32).reshape(1, hdim))

    out = out3.reshape(hdim, n_pad)[:, :n].T
    return out


# speedup vs baseline: 1.0267x; 1.0267x over previous
import functools

import jax
import jax.numpy as jnp
from jax import lax
from jax.experimental import pallas as pl
from jax.experimental.pallas import tpu as pltpu
from jax.experimental.pallas import tpu_sc as plsc

_NC = 2
_NS = 16
_NW = _NC * _NS
_C = 3200


def _rsqrt16(d):
    i = lax.bitcast_convert_type(d, jnp.int32)
    i = jnp.int32(0x5F3759DF) - lax.shift_right_logical(i, 1)
    y = lax.bitcast_convert_type(i, jnp.float32)
    for _ in range(3):
        y = y * (1.5 - 0.5 * d * y * y)
    return y


def _make_pass(n_pad, e_pad, mode):
    k_total = e_pad // (_NW * _C)
    assert k_total % 2 == 0
    k2 = k_total // 2
    ew = k_total * _C
    slc = n_pad // _NS
    has_g = mode != "deg"
    nsc = 2 if mode == "pm" else 1
    n_tab_in = {"deg": 0, "sum": 2, "pm": 3}[mode]

    def buf_set():
        t = []
        if has_g:
            t.append(pltpu.VMEM((_C,), jnp.int32))
        t.append(pltpu.VMEM((_C,), jnp.int32))
        t.append(pltpu.VMEM((_C,), jnp.float32))
        if has_g:
            t.append(pltpu.VMEM((_C,), jnp.float32))
            for _ in range(nsc):
                t.append(pltpu.VMEM((_C,), jnp.float32))
        return t

    nbuf = len(buf_set())
    naux = {"deg": 0, "sum": 3, "pm": 5}[mode]
    scratch_types = buf_set() + buf_set()
    scratch_types.append(pltpu.VMEM((slc,), jnp.float32))
    for _ in range(naux):
        scratch_types.append(pltpu.VMEM((slc,), jnp.float32))
    if has_g:
        scratch_types.append(pltpu.VMEM_SHARED((n_pad,), jnp.float32))
    for _ in range(nsc):
        scratch_types.append(pltpu.VMEM_SHARED((n_pad,), jnp.float32))
    scratch_types += [pltpu.SemaphoreType.DMA] * 4

    out_sds = jax.ShapeDtypeStruct((_NC * n_pad,), jnp.float32)
    out_type = tuple(out_sds for _ in range(nsc)) if nsc > 1 else out_sds

    @functools.partial(
        pl.kernel,
        mesh=plsc.VectorSubcoreMesh(core_axis_name="c", subcore_axis_name="s"),
        out_type=out_type,
        scratch_types=scratch_types,
    )
    def edge_pass(*refs):
        n_in = 3 + n_tab_in if has_g else 2
        ins = refs[:n_in]
        edge_ins = ins[n_tab_in:]
        if has_g:
            src_hbm, dst_hbm, w_hbm = edge_ins
        else:
            dst_hbm, w_hbm = edge_ins
        outs = refs[n_in:n_in + nsc]
        sc = refs[n_in + nsc:]
        bufa = sc[:nbuf]
        bufb = sc[nbuf:2 * nbuf]
        tmp_v = sc[2 * nbuf]
        aux = sc[2 * nbuf + 1:2 * nbuf + 1 + naux]
        pos = 2 * nbuf + 1 + naux
        if has_g:
            tab_sh = sc[pos]
            pos += 1
        accs = sc[pos:pos + nsc]
        in_sema, in_semb, sc_sema, sc_semb = sc[pos + nsc:pos + nsc + 4]

        def parts(buf):
            if has_g:
                return {"src": buf[0], "dst": buf[1], "w": buf[2],
                        "g": buf[3], "v": buf[4:4 + nsc]}
            return {"dst": buf[0], "w": buf[1], "v": [buf[1]]}

        ba, bb = parts(bufa), parts(bufb)

        c = lax.axis_index("c")
        s = lax.axis_index("s")
        wid = c * _NS + s
        base_w = wid * ew

        def zero_body(j, carry):
            tmp_v[pl.ds(j * 16, 16)] = jnp.zeros((16,), jnp.float32)
            return carry

        lax.fori_loop(0, slc // 16, zero_body, 0)
        for acc in accs:
            pltpu.sync_copy(tmp_v, acc.at[pl.ds(s * slc, slc)])

        if mode == "sum":
            deg_hbm, x_hbm = ins[0], ins[1]
            qa, qb, qc = aux
            pltpu.sync_copy(deg_hbm.at[pl.ds(s * slc, slc)], qa)
            pltpu.sync_copy(deg_hbm.at[pl.ds(n_pad + s * slc, slc)], qb)
            pltpu.sync_copy(x_hbm.at[pl.ds(s * slc, slc)], qc)

            def tab_body(j, carry):
                o = j * 16
                deg = qa[pl.ds(o, 16)] + qb[pl.ds(o, 16)] + 1.0
                tmp_v[pl.ds(o, 16)] = qc[pl.ds(o, 16)] * _rsqrt16(deg)
                return carry

            lax.fori_loop(0, slc // 16, tab_body, 0)
            pltpu.sync_copy(tmp_v, tab_sh.at[pl.ds(s * slc, slc)])
        elif mode == "pm":
            deg_hbm, sacc_hbm, x_hbm = ins[0], ins[1], ins[2]
            qa, qb, qc, qd, qe = aux
            pltpu.sync_copy(deg_hbm.at[pl.ds(s * slc, slc)], qa)
            pltpu.sync_copy(deg_hbm.at[pl.ds(n_pad + s * slc, slc)], qb)
            pltpu.sync_copy(x_hbm.at[pl.ds(s * slc, slc)], qc)
            pltpu.sync_copy(sacc_hbm.at[pl.ds(s * slc, slc)], qd)
            pltpu.sync_copy(sacc_hbm.at[pl.ds(n_pad + s * slc, slc)], qe)

            def tab_body(j, carry):
                o = j * 16
                deg = qa[pl.ds(o, 16)] + qb[pl.ds(o, 16)] + 1.0
                y = _rsqrt16(deg)
                acc16 = qd[pl.ds(o, 16)] + qe[pl.ds(o, 16)]
                tmp_v[pl.ds(o, 16)] = (
                    y * y * (acc16 + qc[pl.ds(o, 16)] * y))
                return carry

            lax.fori_loop(0, slc // 16, tab_body, 0)
            pltpu.sync_copy(tmp_v, tab_sh.at[pl.ds(s * slc, slc)])
        plsc.subcore_barrier()

        def start_in(k, b, sem):
            base = base_w + k * _C
            if has_g:
                pltpu.async_copy(src_hbm.at[pl.ds(base, _C)], b["src"], sem)
            pltpu.async_copy(dst_hbm.at[pl.ds(base, _C)], b["dst"], sem)
            pltpu.async_copy(w_hbm.at[pl.ds(base, _C)], b["w"], sem)

        def wait_in(b, sem):
            if has_g:
                pltpu.make_async_copy(
                    src_hbm.at[pl.ds(0, _C)], b["src"], sem).wait()
            pltpu.make_async_copy(dst_hbm.at[pl.ds(0, _C)], b["dst"], sem).wait()
            pltpu.make_async_copy(w_hbm.at[pl.ds(0, _C)], b["w"], sem).wait()

        def compute(b):
            if mode == "sum":
                def body(j, carry):
                    o = j * 16
                    b["v"][0][pl.ds(o, 16)] = (
                        b["w"][pl.ds(o, 16)] * b["g"][pl.ds(o, 16)])
                    return carry
                lax.fori_loop(0, _C // 16, body, 0)
            elif mode == "pm":
                def body(j, carry):
                    o = j * 16
                    g = b["g"][pl.ds(o, 16)]
                    w16 = b["w"][pl.ds(o, 16)]
                    b["v"][0][pl.ds(o, 16)] = w16 * jnp.maximum(g, 0.0)
                    b["v"][1][pl.ds(o, 16)] = w16 * jnp.minimum(g, 0.0)
                    return carry
                lax.fori_loop(0, _C // 16, body, 0)

        def start_sc(b, sem):
            for v, acc in zip(b["v"], accs):
                pltpu.async_copy(v, acc.at[b["dst"]], sem, add=True)

        def wait_sc(b, sem):
            for v, acc in zip(b["v"], accs):
                pltpu.make_async_copy(v, acc.at[b["dst"]], sem).wait()

        def process(b, in_sem, sc_sem):
            wait_in(b, in_sem)
            if has_g:
                pltpu.sync_copy(tab_sh.at[b["src"]], b["g"])
            compute(b)
            start_sc(b, sc_sem)

        start_in(0, ba, in_sema)
        start_in(1, bb, in_semb)

        def pair_body(kk, carry):
            process(ba, in_sema, sc_sema)
            process(bb, in_semb, sc_semb)

            @pl.when(kk + 1 < k2)
            def _():
                wait_sc(ba, sc_sema)
                start_in(2 * kk + 2, ba, in_sema)
                wait_sc(bb, sc_semb)
                start_in(2 * kk + 3, bb, in_semb)

            return carry

        lax.fori_loop(0, k2, pair_body, 0)
        wait_sc(ba, sc_sema)
        wait_sc(bb, sc_semb)
        plsc.subcore_barrier()
        for acc, out in zip(accs, outs):
            pltpu.sync_copy(acc.at[pl.ds(s * slc, slc)], tmp_v)
            pltpu.sync_copy(tmp_v, out.at[pl.ds(c * n_pad + s * slc, slc)])

    return edge_pass


def _final_kernel(dp_ref, sacc_ref, cp_ref, cm_ref, x_ref,
                  w1_ref, w2_ref, b2_ref, out_ref):
    hdim = w2_ref.shape[1]
    deg = dp_ref[0] + dp_ref[1] + 1.0
    d = lax.rsqrt(deg)
    xd = x_ref[...] * d
    gs = d * d * (sacc_ref[0] + sacc_ref[1] + xd)
    alpha = d * (cp_ref[0] + cp_ref[1] + jnp.maximum(gs, 0.0))
    beta = d * (cm_ref[0] + cm_ref[1] + jnp.minimum(gs, 0.0))
    for h in range(hdim):
        up_h = 0.0
        um_h = 0.0
        for k in range(w2_ref.shape[0]):
            w1k = w1_ref[0, k]
            w2kh = w2_ref[k, h]
            up_h = up_h + jnp.maximum(w1k, 0.0) * w2kh
            um_h = um_h + jnp.minimum(w1k, 0.0) * w2kh
        out_ref[h] = alpha * up_h + beta * um_h + b2_ref[0, h]


def kernel(x, edge_index, edge_weight, W1, b1, W2, b2):
    n = x.shape[0]
    e = edge_weight.shape[0]
    hdim = W2.shape[1]

    n_pad = -(-n // 128) * 128
    group = _NW * _C * 2
    e_pad = -(-e // group) * group
    rows = n_pad // 128

    src = edge_index[0].astype(jnp.int32)
    dst = edge_index[1].astype(jnp.int32)
    w = edge_weight.astype(jnp.float32)
    npad_e = e_pad - e
    if npad_e:
        pad_idx = jnp.arange(npad_e, dtype=jnp.int32) % jnp.int32(n)
        src = jnp.concatenate([src, pad_idx])
        dst = jnp.concatenate([dst, pad_idx])
        w = jnp.concatenate([w, jnp.zeros((npad_e,), jnp.float32)])

    x1 = jnp.pad(x[:, 0].astype(jnp.float32), (0, n_pad - n))

    deg_p = _make_pass(n_pad, e_pad, "deg")(dst, w)

    sacc = _make_pass(n_pad, e_pad, "sum")(deg_p, x1, src, dst, w)

    cp, cm = _make_pass(n_pad, e_pad, "pm")(deg_p, sacc, x1, src, dst, w)

    out3 = pl.pallas_call(
        _final_kernel,
        in_specs=[
            pl.BlockSpec(memory_space=pltpu.VMEM),
            pl.BlockSpec(memory_space=pltpu.VMEM),
            pl.BlockSpec(memory_space=pltpu.VMEM),
            pl.BlockSpec(memory_space=pltpu.VMEM),
            pl.BlockSpec(memory_space=pltpu.VMEM),
            pl.BlockSpec(memory_space=pltpu.SMEM),
            pl.BlockSpec(memory_space=pltpu.SMEM),
            pl.BlockSpec(memory_space=pltpu.SMEM),
        ],
        out_shape=jax.ShapeDtypeStruct((hdim, rows, 128), jnp.float32),
    )(deg_p.reshape(_NC, rows, 128), sacc.reshape(_NC, rows, 128),
      cp.reshape(_NC, rows, 128), cm.reshape(_NC, rows, 128),
      x1.reshape(rows, 128),
      W1.astype(jnp.float32), W2.astype(jnp.float32),
      b2.astype(jnp.float32).reshape(1, hdim))

    out = out3.reshape(hdim, n_pad)[:, :n].T
    return out


# speedup vs baseline: 1.1152x; 1.0861x over previous
import functools

import jax
import jax.numpy as jnp
from jax import lax
from jax.experimental import pallas as pl
from jax.experimental.pallas import tpu as pltpu
from jax.experimental.pallas import tpu_sc as plsc

_NC = 2
_NS = 16
_NW = _NC * _NS
_C = 5000


def _rsqrt16(d):
    i = lax.bitcast_convert_type(d, jnp.int32)
    i = jnp.int32(0x5F3759DF) - lax.shift_right_logical(i, 1)
    y = lax.bitcast_convert_type(i, jnp.float32)
    for _ in range(3):
        y = y * (1.5 - 0.5 * d * y * y)
    return y


def _make_pass(n_pad, e_pad, mode):
    k_total = e_pad // (_NW * _C)
    assert k_total % 2 == 0
    k2 = k_total // 2
    ew = k_total * _C
    slc = n_pad // _NS
    has_g = mode != "deg"
    nsc = 2 if mode == "pm" else 1
    n_tab_in = {"deg": 0, "sum": 2, "pm": 3}[mode]

    def buf_set():
        t = []
        if has_g:
            t.append(pltpu.VMEM((_C,), jnp.int32))
        t.append(pltpu.VMEM((_C,), jnp.int32))
        t.append(pltpu.VMEM((_C,), jnp.float32))
        if has_g:
            t.append(pltpu.VMEM((_C,), jnp.float32))
            for _ in range(nsc):
                t.append(pltpu.VMEM((_C,), jnp.float32))
        return t

    nbuf = len(buf_set())
    naux = {"deg": 0, "sum": 3, "pm": 5}[mode]
    scratch_types = buf_set() + buf_set()
    scratch_types.append(pltpu.VMEM((slc,), jnp.float32))
    for _ in range(naux):
        scratch_types.append(pltpu.VMEM((slc,), jnp.float32))
    if has_g:
        scratch_types.append(pltpu.VMEM_SHARED((n_pad,), jnp.float32))
    for _ in range(nsc):
        scratch_types.append(pltpu.VMEM_SHARED((n_pad,), jnp.float32))
    scratch_types += [pltpu.SemaphoreType.DMA] * 4

    out_sds = jax.ShapeDtypeStruct((_NC * n_pad,), jnp.float32)
    out_type = tuple(out_sds for _ in range(nsc)) if nsc > 1 else out_sds

    @functools.partial(
        pl.kernel,
        mesh=plsc.VectorSubcoreMesh(core_axis_name="c", subcore_axis_name="s"),
        out_type=out_type,
        scratch_types=scratch_types,
    )
    def edge_pass(*refs):
        n_in = 2 + n_tab_in
        ins = refs[:n_in]
        ei_hbm, w_hbm = ins[n_tab_in:]
        outs = refs[n_in:n_in + nsc]
        sc = refs[n_in + nsc:]
        bufa = sc[:nbuf]
        bufb = sc[nbuf:2 * nbuf]
        tmp_v = sc[2 * nbuf]
        aux = sc[2 * nbuf + 1:2 * nbuf + 1 + naux]
        pos = 2 * nbuf + 1 + naux
        if has_g:
            tab_sh = sc[pos]
            pos += 1
        accs = sc[pos:pos + nsc]
        in_sema, in_semb, sc_sema, sc_semb = sc[pos + nsc:pos + nsc + 4]

        def parts(buf):
            if has_g:
                return {"src": buf[0], "dst": buf[1], "w": buf[2],
                        "g": buf[3], "v": buf[4:4 + nsc]}
            return {"dst": buf[0], "w": buf[1], "v": [buf[1]]}

        ba, bb = parts(bufa), parts(bufb)

        c = lax.axis_index("c")
        s = lax.axis_index("s")
        wid = c * _NS + s
        base_w = wid * ew

        def zero_body(j, carry):
            tmp_v[pl.ds(j * 16, 16)] = jnp.zeros((16,), jnp.float32)
            return carry

        lax.fori_loop(0, slc // 16, zero_body, 0)
        for acc in accs:
            pltpu.sync_copy(tmp_v, acc.at[pl.ds(s * slc, slc)])

        if mode == "sum":
            deg_hbm, x_hbm = ins[0], ins[1]
            qa, qb, qc = aux
            pltpu.sync_copy(deg_hbm.at[pl.ds(s * slc, slc)], qa)
            pltpu.sync_copy(deg_hbm.at[pl.ds(n_pad + s * slc, slc)], qb)
            pltpu.sync_copy(x_hbm.at[pl.ds(s * slc, slc)], qc)

            def tab_body(j, carry):
                o = j * 16
                deg = qa[pl.ds(o, 16)] + qb[pl.ds(o, 16)] + 1.0
                tmp_v[pl.ds(o, 16)] = qc[pl.ds(o, 16)] * _rsqrt16(deg)
                return carry

            lax.fori_loop(0, slc // 16, tab_body, 0)
            pltpu.sync_copy(tmp_v, tab_sh.at[pl.ds(s * slc, slc)])
        elif mode == "pm":
            deg_hbm, sacc_hbm, x_hbm = ins[0], ins[1], ins[2]
            qa, qb, qc, qd, qe = aux
            pltpu.sync_copy(deg_hbm.at[pl.ds(s * slc, slc)], qa)
            pltpu.sync_copy(deg_hbm.at[pl.ds(n_pad + s * slc, slc)], qb)
            pltpu.sync_copy(x_hbm.at[pl.ds(s * slc, slc)], qc)
            pltpu.sync_copy(sacc_hbm.at[pl.ds(s * slc, slc)], qd)
            pltpu.sync_copy(sacc_hbm.at[pl.ds(n_pad + s * slc, slc)], qe)

            def tab_body(j, carry):
                o = j * 16
                deg = qa[pl.ds(o, 16)] + qb[pl.ds(o, 16)] + 1.0
                y = _rsqrt16(deg)
                acc16 = qd[pl.ds(o, 16)] + qe[pl.ds(o, 16)]
                tmp_v[pl.ds(o, 16)] = (
                    y * y * (acc16 + qc[pl.ds(o, 16)] * y))
                return carry

            lax.fori_loop(0, slc // 16, tab_body, 0)
            pltpu.sync_copy(tmp_v, tab_sh.at[pl.ds(s * slc, slc)])
        plsc.subcore_barrier()

        def start_in(k, b, sem):
            base = base_w + k * _C
            if has_g:
                pltpu.async_copy(ei_hbm.at[pl.ds(base, _C)], b["src"], sem)
            pltpu.async_copy(ei_hbm.at[pl.ds(e_pad + base, _C)], b["dst"], sem)
            pltpu.async_copy(w_hbm.at[pl.ds(base, _C)], b["w"], sem)

        def wait_in(b, sem):
            if has_g:
                pltpu.make_async_copy(
                    ei_hbm.at[pl.ds(0, _C)], b["src"], sem).wait()
            pltpu.make_async_copy(ei_hbm.at[pl.ds(0, _C)], b["dst"], sem).wait()
            pltpu.make_async_copy(w_hbm.at[pl.ds(0, _C)], b["w"], sem).wait()

        def compute(b):
            if mode == "sum":
                def body(j, carry):
                    o = j * 16
                    b["v"][0][pl.ds(o, 16)] = (
                        b["w"][pl.ds(o, 16)] * b["g"][pl.ds(o, 16)])
                    return carry
                lax.fori_loop(0, _C // 16, body, 0)
            elif mode == "pm":
                def body(j, carry):
                    o = j * 16
                    g = b["g"][pl.ds(o, 16)]
                    w16 = b["w"][pl.ds(o, 16)]
                    b["v"][0][pl.ds(o, 16)] = w16 * jnp.maximum(g, 0.0)
                    b["v"][1][pl.ds(o, 16)] = w16 * jnp.minimum(g, 0.0)
                    return carry
                lax.fori_loop(0, _C // 16, body, 0)

        def start_sc(b, sem):
            for v, acc in zip(b["v"], accs):
                pltpu.async_copy(v, acc.at[b["dst"]], sem, add=True)

        def wait_sc(b, sem):
            for v, acc in zip(b["v"], accs):
                pltpu.make_async_copy(v, acc.at[b["dst"]], sem).wait()

        def process(b, in_sem, sc_sem):
            wait_in(b, in_sem)
            if has_g:
                pltpu.sync_copy(tab_sh.at[b["src"]], b["g"])
            compute(b)
            start_sc(b, sc_sem)

        start_in(0, ba, in_sema)
        start_in(1, bb, in_semb)

        def pair_body(kk, carry):
            process(ba, in_sema, sc_sema)
            process(bb, in_semb, sc_semb)

            @pl.when(kk + 1 < k2)
            def _():
                wait_sc(ba, sc_sema)
                start_in(2 * kk + 2, ba, in_sema)
                wait_sc(bb, sc_semb)
                start_in(2 * kk + 3, bb, in_semb)

            return carry

        lax.fori_loop(0, k2, pair_body, 0)
        wait_sc(ba, sc_sema)
        wait_sc(bb, sc_semb)
        plsc.subcore_barrier()
        for acc, out in zip(accs, outs):
            pltpu.sync_copy(acc.at[pl.ds(s * slc, slc)], tmp_v)
            pltpu.sync_copy(tmp_v, out.at[pl.ds(c * n_pad + s * slc, slc)])

    return edge_pass


def _ab_kernel(dp_ref, sacc_ref, cp_ref, cm_ref, x_ref, ab_ref):
    deg = dp_ref[0] + dp_ref[1] + 1.0
    d = lax.rsqrt(deg)
    xd = x_ref[...] * d
    gs = d * d * (sacc_ref[0] + sacc_ref[1] + xd)
    ab_ref[0] = d * (cp_ref[0] + cp_ref[1] + jnp.maximum(gs, 0.0))
    ab_ref[1] = d * (cm_ref[0] + cm_ref[1] + jnp.minimum(gs, 0.0))


def _mm_kernel(ab_ref, w1_ref, w2_ref, b2_ref, out_ref):
    n = out_ref.shape[0]
    w1 = w1_ref[...]
    w2 = w2_ref[...]
    up = jnp.dot(jnp.maximum(w1, 0.0), w2, preferred_element_type=jnp.float32)
    um = jnp.dot(jnp.minimum(w1, 0.0), w2, preferred_element_type=jnp.float32)
    u = jnp.concatenate([up, um], axis=0)
    res = lax.dot_general(ab_ref[...], u, (((0,), (0,)), ((), ())),
                          preferred_element_type=jnp.float32)
    out_ref[...] = res[:n, :] + b2_ref[...]


def kernel(x, edge_index, edge_weight, W1, b1, W2, b2):
    n = x.shape[0]
    e = edge_weight.shape[0]
    hdim = W2.shape[1]

    n_pad = -(-n // 128) * 128
    group = _NW * _C * 2
    e_pad = -(-e // group) * group
    rows = n_pad // 128

    ei = edge_index.astype(jnp.int32)
    w = edge_weight.astype(jnp.float32)
    npad_e = e_pad - e
    if npad_e:
        pad_idx = jnp.arange(npad_e, dtype=jnp.int32) % jnp.int32(n)
        ei_flat = jnp.concatenate(
            [ei[0], pad_idx, ei[1], pad_idx])
        w = jnp.concatenate([w, jnp.zeros((npad_e,), jnp.float32)])
    else:
        ei_flat = ei.reshape(2 * e_pad)

    x1 = jnp.pad(x[:, 0].astype(jnp.float32), (0, n_pad - n))

    deg_p = _make_pass(n_pad, e_pad, "deg")(ei_flat, w)

    sacc = _make_pass(n_pad, e_pad, "sum")(deg_p, x1, ei_flat, w)

    cp, cm = _make_pass(n_pad, e_pad, "pm")(deg_p, sacc, x1, ei_flat, w)

    ab = pl.pallas_call(
        _ab_kernel,
        out_shape=jax.ShapeDtypeStruct((2, rows, 128), jnp.float32),
    )(deg_p.reshape(_NC, rows, 128), sacc.reshape(_NC, rows, 128),
      cp.reshape(_NC, rows, 128), cm.reshape(_NC, rows, 128),
      x1.reshape(rows, 128))

    out = pl.pallas_call(
        _mm_kernel,
        out_shape=jax.ShapeDtypeStruct((n, hdim), jnp.float32),
    )(ab.reshape(2, n_pad), W1.astype(jnp.float32),
      W2.astype(jnp.float32), b2.astype(jnp.float32).reshape(1, hdim))
    return out
